# Initial kernel scaffold; baseline (speedup 1.0000x reference)
import functools

import jax
import jax.numpy as jnp
from jax import lax
from jax.experimental import pallas as pl
from jax.experimental.pallas import tpu as pltpu
from jax.experimental.pallas import tpu_sc as plsc

NNZ = 327680
BATCH = 16384
NUM_VALUES = 100001
NUM_FEATURES = 64

NC = 2
NS = 16
NW = NC * NS
L = 16

IB = 128
ROWS_TOTAL = NNZ // IB
ROWS_PER_W = ROWS_TOTAL // NW
JROWS = 4
CHUNK = JROWS * IB
NCHUNK = ROWS_PER_W // JROWS
ROWS_PER_TILE = BATCH // NS
CW = 16


def _sc_body(matrix, colh, rowh, cnth, out_e, out_c,
             gbuf, cstage, col_v, row_v, cnt_v, acc_e, acc_c):
    cid = lax.axis_index("c")
    sid = lax.axis_index("s")
    w = sid * NC + cid

    zero16 = jnp.zeros((L,), jnp.float32)

    def _zero(i, _):
        for q in range(NUM_FEATURES // L):
            gbuf[i, pl.ds(q * L, L)] = zero16
        cstage[i, pl.ds(0, L)] = zero16
        return 0
    lax.fori_loop(0, CHUNK, _zero, 0)

    for k in range(ROWS_PER_TILE // CHUNK):
        base = sid * ROWS_PER_TILE + k * CHUNK
        pltpu.sync_copy(gbuf, acc_e.at[pl.ds(base, CHUNK)])
        pltpu.sync_copy(cstage, acc_c.at[pl.ds(base, CHUNK)])
    plsc.subcore_barrier()

    def chunk_body(t, _):
        base = w * ROWS_PER_W + t * JROWS
        pltpu.sync_copy(colh.at[pl.ds(base, JROWS)], col_v)
        pltpu.sync_copy(rowh.at[pl.ds(base, JROWS)], row_v)
        pltpu.sync_copy(cnth.at[pl.ds(base, JROWS)], cnt_v)
        for j in range(JROWS):
            pltpu.sync_copy(matrix.at[col_v.at[j]],
                            gbuf.at[pl.ds(j * IB, IB)])
        for j in range(JROWS):
            def scale_body(i, _):
                r = j * IB + i
                c = cnt_v[j, i]
                for q in range(NUM_FEATURES // L):
                    gbuf[r, pl.ds(q * L, L)] = gbuf[r, pl.ds(q * L, L)] * c
                cstage[r, pl.ds(0, L)] = c * jnp.ones((L,), jnp.float32)
                return 0
            lax.fori_loop(0, IB, scale_body, 0)
        for j in range(JROWS):
            pltpu.sync_copy(gbuf.at[pl.ds(j * IB, IB)],
                            acc_e.at[row_v.at[j]], add=True)
            pltpu.sync_copy(cstage.at[pl.ds(j * IB, IB)],
                            acc_c.at[row_v.at[j]], add=True)
        return 0
    lax.fori_loop(0, NCHUNK, chunk_body, 0)

    plsc.subcore_barrier()
    base = sid * ROWS_PER_TILE
    pltpu.sync_copy(acc_e.at[pl.ds(base, ROWS_PER_TILE)],
                    out_e.at[cid].at[pl.ds(base, ROWS_PER_TILE)])
    pltpu.sync_copy(acc_c.at[pl.ds(base, ROWS_PER_TILE)],
                    out_c.at[cid].at[pl.ds(base, ROWS_PER_TILE)])


def _combine_body(pe_ref, pc_ref, o_ref):
    e = pe_ref[0] + pe_ref[1]
    t = pc_ref[0, :, 0:1] + pc_ref[1, :, 0:1]
    o_ref[...] = e / t


@jax.jit
def kernel(matrix, counts, row_ids, col_ids):
    colh = col_ids.astype(jnp.int32).reshape(ROWS_TOTAL, IB)
    rowh = row_ids.astype(jnp.int32).reshape(ROWS_TOTAL, IB)
    cnth = counts.reshape(ROWS_TOTAL, IB)

    mesh = plsc.VectorSubcoreMesh(core_axis_name="c", subcore_axis_name="s")
    sc = pl.kernel(
        _sc_body,
        out_type=[
            jax.ShapeDtypeStruct((NC, BATCH, NUM_FEATURES), jnp.float32),
            jax.ShapeDtypeStruct((NC, BATCH, CW), jnp.float32),
        ],
        mesh=mesh,
        scratch_types=[
            pltpu.VMEM((CHUNK, NUM_FEATURES), jnp.float32),
            pltpu.VMEM((CHUNK, CW), jnp.float32),
            pltpu.VMEM((JROWS, IB), jnp.int32),
            pltpu.VMEM((JROWS, IB), jnp.int32),
            pltpu.VMEM((JROWS, IB), jnp.float32),
            pltpu.VMEM_SHARED((BATCH, NUM_FEATURES), jnp.float32),
            pltpu.VMEM_SHARED((BATCH, CW), jnp.float32),
        ],
    )
    part_e, part_c = sc(matrix, colh, rowh, cnth)

    rows_blk = 1024
    out = pl.pallas_call(
        _combine_body,
        grid=(BATCH // rows_blk,),
        in_specs=[
            pl.BlockSpec((NC, rows_blk, NUM_FEATURES), lambda i: (0, i, 0)),
            pl.BlockSpec((NC, rows_blk, CW), lambda i: (0, i, 0)),
        ],
        out_specs=pl.BlockSpec((rows_blk, NUM_FEATURES), lambda i: (i, 0)),
        out_shape=jax.ShapeDtypeStruct((BATCH, NUM_FEATURES), jnp.float32),
    )(part_e, part_c)
    return out


# speedup vs baseline: 5.5803x; 5.5803x over previous
import functools

import jax
import jax.numpy as jnp
from jax import lax
from jax.experimental import pallas as pl
from jax.experimental.pallas import tpu as pltpu
from jax.experimental.pallas import tpu_sc as plsc

NNZ = 327680
BATCH = 16384
NUM_VALUES = 100001
NUM_FEATURES = 64

NC = 2
NS = 16
NW = NC * NS
L = 16

IB = 128
ROWS_TOTAL = NNZ // IB
ROWS_PER_W = ROWS_TOTAL // NW
JROWS = 4
CHUNK = JROWS * IB
NCHUNK = ROWS_PER_W // JROWS
ROWS_PER_TILE = BATCH // NS
CW = 16


def _sc_body(matrix, colh, rowh, cnth, out_e, out_c,
             gbuf, cstage, col_v, row_v, cnt_v, acc_e, acc_c):
    cid = lax.axis_index("c")
    sid = lax.axis_index("s")
    w = sid * NC + cid

    zero16 = jnp.zeros((L,), jnp.float32)

    def _zero(i, _):
        for q in range(NUM_FEATURES // L):
            gbuf[i, pl.ds(q * L, L)] = zero16
        cstage[i, pl.ds(0, L)] = zero16
        return 0
    lax.fori_loop(0, CHUNK, _zero, 0)

    for k in range(ROWS_PER_TILE // CHUNK):
        base = sid * ROWS_PER_TILE + k * CHUNK
        pltpu.sync_copy(gbuf, acc_e.at[pl.ds(base, CHUNK)])
        pltpu.sync_copy(cstage, acc_c.at[pl.ds(base, CHUNK)])
    plsc.subcore_barrier()

    def chunk_body(t, _):
        base = w * ROWS_PER_W + t * JROWS
        pltpu.sync_copy(colh.at[pl.ds(base, JROWS)], col_v)
        pltpu.sync_copy(rowh.at[pl.ds(base, JROWS)], row_v)
        pltpu.sync_copy(cnth.at[pl.ds(base, JROWS)], cnt_v)
        for j in range(JROWS):
            pltpu.sync_copy(matrix.at[col_v.at[j]],
                            gbuf.at[pl.ds(j * IB, IB)])
        ones = jnp.ones((L,), jnp.float32)
        for j in range(JROWS):
            def scale_body(g, _):
                cv = cnt_v[j, pl.ds(g * L, L)]
                for e in range(L):
                    r = j * IB + g * L + e
                    c = cv[e] * ones
                    for q in range(NUM_FEATURES // L):
                        gbuf[r, pl.ds(q * L, L)] = (
                            gbuf[r, pl.ds(q * L, L)] * c)
                    cstage[r, pl.ds(0, L)] = c
                return 0
            lax.fori_loop(0, IB // L, scale_body, 0)
        for j in range(JROWS):
            pltpu.sync_copy(gbuf.at[pl.ds(j * IB, IB)],
                            acc_e.at[row_v.at[j]], add=True)
            pltpu.sync_copy(cstage.at[pl.ds(j * IB, IB)],
                            acc_c.at[row_v.at[j]], add=True)
        return 0
    lax.fori_loop(0, NCHUNK, chunk_body, 0)

    plsc.subcore_barrier()
    base = sid * ROWS_PER_TILE
    pltpu.sync_copy(acc_e.at[pl.ds(base, ROWS_PER_TILE)],
                    out_e.at[cid].at[pl.ds(base, ROWS_PER_TILE)])
    pltpu.sync_copy(acc_c.at[pl.ds(base, ROWS_PER_TILE)],
                    out_c.at[cid].at[pl.ds(base, ROWS_PER_TILE)])


def _combine_body(pe_ref, pc_ref, o_ref):
    e = pe_ref[0] + pe_ref[1]
    t = pc_ref[0, :, 0:1] + pc_ref[1, :, 0:1]
    o_ref[...] = e / t


@jax.jit
def kernel(matrix, counts, row_ids, col_ids):
    colh = col_ids.astype(jnp.int32).reshape(ROWS_TOTAL, IB)
    rowh = row_ids.astype(jnp.int32).reshape(ROWS_TOTAL, IB)
    cnth = counts.reshape(ROWS_TOTAL, IB)

    mesh = plsc.VectorSubcoreMesh(core_axis_name="c", subcore_axis_name="s")
    sc = pl.kernel(
        _sc_body,
        out_type=[
            jax.ShapeDtypeStruct((NC, BATCH, NUM_FEATURES), jnp.float32),
            jax.ShapeDtypeStruct((NC, BATCH, CW), jnp.float32),
        ],
        mesh=mesh,
        compiler_params=pltpu.CompilerParams(use_tc_tiling_on_sc=False),
        scratch_types=[
            pltpu.VMEM((CHUNK, NUM_FEATURES), jnp.float32),
            pltpu.VMEM((CHUNK, CW), jnp.float32),
            pltpu.VMEM((JROWS, IB), jnp.int32),
            pltpu.VMEM((JROWS, IB), jnp.int32),
            pltpu.VMEM((JROWS, IB), jnp.float32),
            pltpu.VMEM_SHARED((BATCH, NUM_FEATURES), jnp.float32),
            pltpu.VMEM_SHARED((BATCH, CW), jnp.float32),
        ],
    )
    part_e, part_c = sc(matrix, colh, rowh, cnth)

    rows_blk = 1024
    out = pl.pallas_call(
        _combine_body,
        grid=(BATCH // rows_blk,),
        in_specs=[
            pl.BlockSpec((NC, rows_blk, NUM_FEATURES), lambda i: (0, i, 0)),
            pl.BlockSpec((NC, rows_blk, CW), lambda i: (0, i, 0)),
        ],
        out_specs=pl.BlockSpec((rows_blk, NUM_FEATURES), lambda i: (i, 0)),
        out_shape=jax.ShapeDtypeStruct((BATCH, NUM_FEATURES), jnp.float32),
    )(part_e, part_c)
    return out


# speedup vs baseline: 7.1823x; 1.2871x over previous
import jax
import jax.numpy as jnp
from jax import lax
from jax.experimental import pallas as pl
from jax.experimental.pallas import tpu as pltpu
from jax.experimental.pallas import tpu_sc as plsc

NNZ = 327680
BATCH = 16384
NUM_VALUES = 100001
NUM_FEATURES = 64

NC = 2
NS = 16
NW = NC * NS
L = 16
QF = NUM_FEATURES // L

IB = 128
ROWS_TOTAL = NNZ // IB
NCHUNK = ROWS_TOTAL // NW
NBUF = 4
NIDX = 8
ROWS_PER_TILE = BATCH // NS


def _sc_body(matrix, idxh, out_e, out_c,
             acc_e, acc_c,
             g0, g1, g2, g3, c0, c1, c2, c3,
             i0, i1, i2, i3, i4, i5, i6, i7,
             gs0, gs1, gs2, gs3, ss0, ss1, ss2, ss3,
             is0, is1, is2, is3, is4, is5, is6, is7):
    gb = [g0, g1, g2, g3]
    cs = [c0, c1, c2, c3]
    ib = [i0, i1, i2, i3, i4, i5, i6, i7]
    gsem = [gs0, gs1, gs2, gs3]
    ssem = [ss0, ss1, ss2, ss3]
    isem = [is0, is1, is2, is3, is4, is5, is6, is7]

    cid = lax.axis_index("c")
    sid = lax.axis_index("s")
    w = sid * NC + cid
    base_r = w * NCHUNK

    zero16 = jnp.zeros((L,), jnp.float32)
    ones = jnp.ones((L,), jnp.float32)

    def _zero(i, _):
        for q in range(QF):
            gb[0][i, pl.ds(q * L, L)] = zero16
        return 0
    lax.fori_loop(0, IB, _zero, 0)

    def _zero_c(g, _):
        cs[0][pl.ds(g * L, L)] = zero16
        return 0
    lax.fori_loop(0, IB // L, _zero_c, 0)

    for k in range(ROWS_PER_TILE // IB):
        base = sid * ROWS_PER_TILE + k * IB
        pltpu.sync_copy(gb[0], acc_e.at[pl.ds(base, IB)])
        pltpu.sync_copy(cs[0], acc_c.at[pl.ds(base, IB)])
    plsc.subcore_barrier()

    def fire_idx(t, s):
        pltpu.async_copy(idxh.at[base_r + t], ib[s], isem[s])

    def wait_idx(s):
        pltpu.make_async_copy(idxh.at[0], ib[s], isem[s]).wait()

    def fire_gather(b, s):
        pltpu.async_copy(matrix.at[ib[s].at[0]], gb[b], gsem[b])

    def wait_gather(b):
        pltpu.make_async_copy(matrix.at[ib[0].at[0]], gb[b],
                              gsem[b]).wait()

    def fire_scatter(b, s):
        pltpu.async_copy(gb[b], acc_e.at[ib[s].at[1]], ssem[b], add=True)
        pltpu.async_copy(cs[b], acc_c.at[ib[s].at[1]], ssem[b], add=True)

    def wait_scatter(b):
        pltpu.make_async_copy(gb[b], acc_e.at[ib[0].at[1]], ssem[b]).wait()
        pltpu.make_async_copy(cs[b], acc_c.at[ib[0].at[1]], ssem[b]).wait()

    def scale(b, s):
        def sb(g, _):
            cv = plsc.bitcast(ib[s][2, pl.ds(g * L, L)], jnp.float32)
            r0 = g * L
            cs[b][pl.ds(r0, L)] = cv
            for e in range(L):
                c = cv[e] * ones
                for q in range(QF):
                    gb[b][r0 + e, pl.ds(q * L, L)] = (
                        gb[b][r0 + e, pl.ds(q * L, L)] * c)
            return 0
        lax.fori_loop(0, IB // L, sb, 0)

    fire_idx(0, 0)
    fire_idx(1, 1)
    wait_idx(0)
    fire_gather(0, 0)

    def body(gidx, _):
        for u in range(NIDX):
            t = gidx * NIDX + u
            b = u % NBUF
            nb = (u + 1) % NBUF
            s = u
            ns = (u + 1) % NIDX
            ps = (u + 2) % NIDX

            @pl.when(t >= NBUF - 1)
            def _():
                wait_scatter(nb)

            @pl.when(t + 2 < NCHUNK)
            def _():
                fire_idx(t + 2, ps)

            @pl.when(t + 1 < NCHUNK)
            def _():
                wait_idx(ns)
                fire_gather(nb, ns)

            wait_gather(b)
            scale(b, s)
            fire_scatter(b, s)
        return 0
    lax.fori_loop(0, NCHUNK // NIDX, body, 0)

    for t in range(NCHUNK - NBUF + 1, NCHUNK):
        wait_scatter(t % NBUF)

    plsc.subcore_barrier()
    base = sid * ROWS_PER_TILE
    pltpu.sync_copy(acc_e.at[pl.ds(base, ROWS_PER_TILE)],
                    out_e.at[cid].at[pl.ds(base, ROWS_PER_TILE)])
    pltpu.sync_copy(acc_c.at[pl.ds(base, ROWS_PER_TILE)],
                    out_c.at[cid].at[pl.ds(base, ROWS_PER_TILE)])


def _combine_body(pe_ref, pc_ref, o_ref):
    e = pe_ref[0] + pe_ref[1]
    t = pc_ref[0] + pc_ref[1]
    o_ref[...] = e / t


@jax.jit
def kernel(matrix, counts, row_ids, col_ids):
    colh = col_ids.astype(jnp.int32).reshape(ROWS_TOTAL, 1, IB)
    rowh = row_ids.astype(jnp.int32).reshape(ROWS_TOTAL, 1, IB)
    cnth = lax.bitcast_convert_type(counts, jnp.int32).reshape(
        ROWS_TOTAL, 1, IB)
    idxh = jnp.concatenate([colh, rowh, cnth], axis=1)

    mesh = plsc.VectorSubcoreMesh(core_axis_name="c", subcore_axis_name="s")
    sc = pl.kernel(
        _sc_body,
        out_type=[
            jax.ShapeDtypeStruct((NC, BATCH, NUM_FEATURES), jnp.float32),
            jax.ShapeDtypeStruct((NC, BATCH), jnp.float32),
        ],
        mesh=mesh,
        compiler_params=pltpu.CompilerParams(
            use_tc_tiling_on_sc=False, needs_layout_passes=False),
        scratch_types=(
            [
                pltpu.VMEM_SHARED((BATCH, NUM_FEATURES), jnp.float32),
                pltpu.VMEM_SHARED((BATCH,), jnp.float32),
            ]
            + [pltpu.VMEM((IB, NUM_FEATURES), jnp.float32)] * NBUF
            + [pltpu.VMEM((IB,), jnp.float32)] * NBUF
            + [pltpu.VMEM((3, IB), jnp.int32)] * NIDX
            + [pltpu.SemaphoreType.DMA] * (2 * NBUF + NIDX)
        ),
    )
    part_e, part_c = sc(matrix, idxh)
    part_c = part_c.reshape(NC, BATCH, 1)

    rows_blk = 1024
    out = pl.pallas_call(
        _combine_body,
        grid=(BATCH // rows_blk,),
        in_specs=[
            pl.BlockSpec((NC, rows_blk, NUM_FEATURES), lambda i: (0, i, 0)),
            pl.BlockSpec((NC, rows_blk, 1), lambda i: (0, i, 0)),
        ],
        out_specs=pl.BlockSpec((rows_blk, NUM_FEATURES), lambda i: (i, 0)),
        out_shape=jax.ShapeDtypeStruct((BATCH, NUM_FEATURES), jnp.float32),
    )(part_e, part_c)
    return out
